# baseline (speedup 1.0000x reference)
import dataclasses
import functools

import jax
import jax.numpy as jnp
from jax import lax
from jax.experimental import pallas as pl
from jax.experimental.pallas import tpu as pltpu
from jax.experimental.pallas import tpu_sc as plsc

N = 100000
E = 3200000
B = 16384

N_PAD = 102400
E_PAD = 3276800
NC, NS = 2, 16
NW = NC * NS
EPT = E_PAD // NW
CH = 1024
ROWS = CH // 128
NCHUNK = EPT // CH
E_ALLOC = E_PAD + 2 * CH
SLICE = N_PAD // NS
BPW = B // NW

_mesh = plsc.VectorSubcoreMesh(core_axis_name="c", subcore_axis_name="s")
_f32 = jnp.float32
_i32 = jnp.int32

_cp = pltpu.CompilerParams()
if "needs_layout_passes" in pltpu.CompilerParams.__dataclass_fields__:
    _cp = dataclasses.replace(_cp, needs_layout_passes=False)


@functools.partial(
    pl.kernel,
    compiler_params=_cp,
    out_type=jax.ShapeDtypeStruct((NW, N_PAD), _f32),
    mesh=_mesh,
    scratch_types=(
        [pltpu.VMEM((N_PAD,), _f32)]
        + [pltpu.VMEM((CH,), _i32) for _ in range(2)]
        + [pltpu.VMEM((CH,), _f32) for _ in range(2)]
        + [pltpu.SemaphoreType.DMA for _ in range(2)]
    ),
)
def _sc_deg(dst_hbm, ew_hbm, z_hbm, out_hbm, acc,
            ix0, ix1, v0, v1, sx0, sx1):
    c = lax.axis_index("c")
    s = lax.axis_index("s")
    w = c * NS + s
    IX = (ix0, ix1)
    V = (v0, v1)
    SX = (sx0, sx1)
    ebase = w * EPT

    def issue_in(chunk, bk):
        sl = pl.ds(ebase + chunk * CH, CH)
        pltpu.async_copy(dst_hbm.at[sl], IX[bk], SX[bk])
        pltpu.async_copy(ew_hbm.at[sl], V[bk], SX[bk])

    def wait_in(chunk, bk):
        sl = pl.ds(ebase + chunk * CH, CH)
        pltpu.make_async_copy(dst_hbm.at[sl], IX[bk], SX[bk]).wait()
        pltpu.make_async_copy(ew_hbm.at[sl], V[bk], SX[bk]).wait()

    def process(chunk, bk):
        wait_in(chunk, bk)
        issue_in(chunk + 2, bk)
        ix, v = IX[bk], V[bk]

        @pl.loop(0, CH, step=16)
        def _(g):
            plsc.addupdate_scatter(acc, [ix[pl.ds(g, 16)]], v[pl.ds(g, 16)])

    issue_in(0, 0)
    issue_in(1, 1)
    pltpu.sync_copy(z_hbm, acc)
    process(0, 0)
    process(1, 1)

    @pl.loop(2, NCHUNK, step=2)
    def _(i0):
        for b in range(2):
            process(i0 + b, b)

    wait_in(NCHUNK, 0)
    wait_in(NCHUNK + 1, 1)
    pltpu.sync_copy(acc, out_hbm.at[w])


@functools.partial(
    pl.kernel,
    compiler_params=_cp,
    out_type=jax.ShapeDtypeStruct((NC, 2, N_PAD), _f32),
    mesh=_mesh,
    scratch_types=(
        [pltpu.VMEM((N_PAD,), _i32)]
        + [pltpu.VMEM_SHARED((N_PAD,), _f32) for _ in range(2)]
        + [pltpu.VMEM((CH,), _i32) for _ in range(2)]
        + [pltpu.VMEM((CH,), _f32) for _ in range(2)]
        + [pltpu.VMEM((CH,), _i32) for _ in range(4)]
        + [pltpu.VMEM((CH,), _f32) for _ in range(4)]
        + [pltpu.SemaphoreType.DMA for _ in range(8)]
    ),
)
def _sc_prop(src_hbm, dst_hbm, ew_hbm, hp_hbm, z_hbm, out_hbm,
             tab, acc0, acc1, sb0, sb1, wb0, wb1, ix0, ix1, ix2, ix3,
             m00, m01, m10, m11,
             si0, si1, sx0, sx1, sx2, sx3, ss0, ss1):
    c = lax.axis_index("c")
    s = lax.axis_index("s")
    w = c * NS + s
    SB = (sb0, sb1)
    WB = (wb0, wb1)
    IX = (ix0, ix1, ix2, ix3)
    M0 = (m00, m01)
    M1 = (m10, m11)
    SI = (si0, si1)
    SX = (sx0, sx1, sx2, sx3)
    SS = (ss0, ss1)
    ebase = w * EPT

    def issue_in(chunk, bin_, bix):
        sl = pl.ds(ebase + chunk * CH, CH)
        pltpu.async_copy(src_hbm.at[sl], SB[bin_], SI[bin_])
        pltpu.async_copy(ew_hbm.at[sl], WB[bin_], SI[bin_])
        pltpu.async_copy(dst_hbm.at[sl], IX[bix], SX[bix])

    def wait_in(chunk, bin_, bix):
        sl = pl.ds(ebase + chunk * CH, CH)
        pltpu.make_async_copy(src_hbm.at[sl], SB[bin_], SI[bin_]).wait()
        pltpu.make_async_copy(ew_hbm.at[sl], WB[bin_], SI[bin_]).wait()
        pltpu.make_async_copy(dst_hbm.at[sl], IX[bix], SX[bix]).wait()

    def drain_sc(bin_, bix):
        pltpu.make_async_copy(M0[bin_], acc0.at[IX[bix]], SS[bin_]).wait()
        pltpu.make_async_copy(M1[bin_], acc1.at[IX[bix]], SS[bin_]).wait()

    def issue_sc(bin_, bix):
        pltpu.async_copy(M0[bin_], acc0.at[IX[bix]], SS[bin_], add=True)
        pltpu.async_copy(M1[bin_], acc1.at[IX[bix]], SS[bin_], add=True)

    def process(chunk, bin_, bix, drain):
        wait_in(chunk, bin_, bix)
        if drain:
            drain_sc(bin_, (bix + 2) % 4)
        sb, wb, m0, m1 = SB[bin_], WB[bin_], M0[bin_], M1[bin_]

        @pl.loop(0, CH, step=16)
        def _(g):
            gi = plsc.load_gather(tab, [sb[pl.ds(g, 16)]])
            w16 = wb[pl.ds(g, 16)]
            m0[pl.ds(g, 16)] = plsc.bitcast(
                lax.shift_left(gi, 16), _f32) * w16
            m1[pl.ds(g, 16)] = plsc.bitcast(
                lax.bitwise_and(gi, jnp.int32(-65536)), _f32) * w16

        issue_in(chunk + 2, bin_, (bix + 2) % 4)
        issue_sc(bin_, bix)

    issue_in(0, 0, 0)
    issue_in(1, 1, 1)
    pltpu.sync_copy(z_hbm.at[pl.ds(s * SLICE, SLICE)],
                    acc0.at[pl.ds(s * SLICE, SLICE)])
    pltpu.sync_copy(z_hbm.at[pl.ds(s * SLICE, SLICE)],
                    acc1.at[pl.ds(s * SLICE, SLICE)])
    pltpu.sync_copy(hp_hbm, tab)
    plsc.subcore_barrier()
    process(0, 0, 0, False)
    process(1, 1, 1, False)

    @pl.loop(2, NCHUNK, step=4)
    def _(i0):
        for b in range(4):
            process(i0 + b, b & 1, (2 + b) % 4, True)

    drain_sc(0, 0)
    drain_sc(1, 1)
    wait_in(NCHUNK, 0, 2)
    wait_in(NCHUNK + 1, 1, 3)
    plsc.subcore_barrier()
    pltpu.sync_copy(acc0.at[pl.ds(s * SLICE, SLICE)],
                    out_hbm.at[c, 0, pl.ds(s * SLICE, SLICE)])
    pltpu.sync_copy(acc1.at[pl.ds(s * SLICE, SLICE)],
                    out_hbm.at[c, 1, pl.ds(s * SLICE, SLICE)])


@functools.partial(
    pl.kernel,
    compiler_params=_cp,
    out_type=[jax.ShapeDtypeStruct((B,), _f32) for _ in range(4)],
    mesh=_mesh,
    scratch_types=[
        pltpu.VMEM((N_PAD,), _f32),
        pltpu.VMEM((BPW,), _i32),
        pltpu.VMEM((BPW,), _f32),
    ],
)
def _sc_pairs(x0_hbm, x1_hbm, home_hbm, away_hbm,
              oh0_hbm, oh1_hbm, oa0_hbm, oa1_hbm, tab, ibuf, obuf):
    c = lax.axis_index("c")
    s = lax.axis_index("s")
    w = c * NS + s
    for x_hbm, outs in ((x0_hbm, (oh0_hbm, oa0_hbm)), (x1_hbm, (oh1_hbm, oa1_hbm))):
        pltpu.sync_copy(x_hbm, tab)
        for i_hbm, o_hbm in zip((home_hbm, away_hbm), outs):
            pltpu.sync_copy(i_hbm.at[pl.ds(w * BPW, BPW)], ibuf)

            @pl.loop(0, BPW, step=16)
            def _(g):
                obuf[pl.ds(g, 16)] = plsc.load_gather(tab, [ibuf[pl.ds(g, 16)]])

            pltpu.sync_copy(obuf, o_hbm.at[pl.ds(w * BPW, BPW)])


def _lrelu(x):
    return jnp.where(x >= 0, x, 0.01 * x)


def _pack_pair(h0, h1):
    u0 = lax.bitcast_convert_type(h0.astype(jnp.bfloat16), jnp.uint16)
    u1 = lax.bitcast_convert_type(h1.astype(jnp.bfloat16), jnp.uint16)
    p = u0.astype(jnp.uint32) | (u1.astype(jnp.uint32) << 16)
    return lax.bitcast_convert_type(p, jnp.int32)


def _tc_prep_body(degp_ref, e0_ref, e1_ref, e2_ref, w1_ref,
                  dinv_ref, d2_ref, h0_ref, h1_ref, hp_ref):
    degp = degp_ref[...]
    deg = jnp.sum(degp, axis=0) + 1.0
    dv = lax.rsqrt(deg)
    dinv_ref[...] = dv
    d2_ref[...] = dv * dv
    w1 = w1_ref[...]
    e0, e1, e2 = e0_ref[...], e1_ref[...], e2_ref[...]
    h0 = e0 * w1[0, 0] + e1 * w1[1, 0] + e2 * w1[2, 0]
    h1 = e0 * w1[0, 1] + e1 * w1[1, 1] + e2 * w1[2, 1]
    h0_ref[...] = h0
    h1_ref[...] = h1
    hp_ref[...] = _pack_pair(dv * h0, dv * h1)


def _tc_prep(degp, e0, e1, e2, w1):
    shp = jax.ShapeDtypeStruct((800, 128), _f32)
    shpi = jax.ShapeDtypeStruct((800, 128), _i32)
    return pl.pallas_call(
        _tc_prep_body, out_shape=[shp, shp, shp, shp, shpi])(degp, e0, e1, e2, w1)


def _tc_mid_body(sp_ref, dinv_ref, d2_ref, h0_ref, h1_ref, b_ref, w2_ref,
                 o0_ref, o1_ref, hp_ref):
    sp = sp_ref[...]
    dv = dinv_ref[...]
    d2 = d2_ref[...]
    b = b_ref[...]
    x0 = _lrelu(dv * (sp[0, 0] + sp[1, 0]) + d2 * h0_ref[...] + b[0, 0])
    x1 = _lrelu(dv * (sp[0, 1] + sp[1, 1]) + d2 * h1_ref[...] + b[0, 1])
    w2 = w2_ref[...]
    h20 = x0 * w2[0, 0] + x1 * w2[1, 0]
    h21 = x0 * w2[0, 1] + x1 * w2[1, 1]
    o0_ref[...] = h20
    o1_ref[...] = h21
    hp_ref[...] = _pack_pair(dv * h20, dv * h21)


def _tc_mid(sp, dinv, d2, h0, h1, b1, w2):
    shp = jax.ShapeDtypeStruct((800, 128), _f32)
    shpi = jax.ShapeDtypeStruct((800, 128), _i32)
    return pl.pallas_call(
        _tc_mid_body, out_shape=[shp, shp, shpi])(sp, dinv, d2, h0, h1, b1, w2)


def _tc_post_body(sp_ref, dinv_ref, d2_ref, h0_ref, h1_ref, b_ref,
                  o0_ref, o1_ref):
    sp = sp_ref[...]
    dv = dinv_ref[...]
    d2 = d2_ref[...]
    b = b_ref[...]
    o0_ref[...] = _lrelu(dv * (sp[0, 0] + sp[1, 0]) + d2 * h0_ref[...] + b[0, 0])
    o1_ref[...] = _lrelu(dv * (sp[0, 1] + sp[1, 1]) + d2 * h1_ref[...] + b[0, 1])


def _tc_post(sp, dinv, d2, h0, h1, b2):
    shp = jax.ShapeDtypeStruct((800, 128), _f32)
    return pl.pallas_call(
        _tc_post_body, out_shape=[shp, shp])(sp, dinv, d2, h0, h1, b2)


def _tc_head_body(gh0_ref, gh1_ref, ga0_ref, ga1_ref,
                  wl1_ref, bl1_ref, wl3_ref, bl3_ref, o0_ref, o1_ref, o2_ref):
    xs = (gh0_ref[...], gh1_ref[...], ga0_ref[...], ga1_ref[...])
    wl1, bl1 = wl1_ref[...], bl1_ref[...]
    ys = []
    for m in range(6):
        acc = bl1[0, m]
        for k in range(4):
            acc = acc + xs[k] * wl1[k, m]
        ys.append(_lrelu(acc))
    wl3, bl3 = wl3_ref[...], bl3_ref[...]
    outs = (o0_ref, o1_ref, o2_ref)
    for t in range(3):
        acc = bl3[0, t]
        for m in range(6):
            acc = acc + ys[m] * wl3[m, t]
        y = _lrelu(acc)
        mx = jnp.max(y)
        lse = jnp.log(jnp.sum(jnp.exp(y - mx))) + mx
        outs[t][...] = y - lse


def _tc_head(gh0, gh1, ga0, ga1, wl1, bl1, wl3, bl3):
    shp = jax.ShapeDtypeStruct((128, 128), _f32)
    return pl.pallas_call(
        _tc_head_body,
        out_shape=[shp, shp, shp])(gh0, gh1, ga0, ga1, wl1, bl1, wl3, bl3)


def kernel(edge_index, edge_weight, home, away, emb,
           W1, b1, W2, b2, Wl1, bl1, Wl3, bl3):
    src = edge_index[0].astype(_i32)
    dst = edge_index[1].astype(_i32)
    pad = E_ALLOC - E
    padi = jnp.full((pad,), N_PAD - 1, _i32)
    srcp = jnp.concatenate([src, padi])
    dstp = jnp.concatenate([dst, padi])
    ewp = jnp.concatenate([edge_weight.astype(_f32), jnp.zeros((pad,), _f32)])
    zin = jnp.zeros((N_PAD,), _f32)

    embp = jnp.pad(emb.astype(_f32), ((0, N_PAD - N), (0, 0)))
    e0 = embp[:, 0].reshape(800, 128)
    e1 = embp[:, 1].reshape(800, 128)
    e2 = embp[:, 2].reshape(800, 128)

    degp = _sc_deg(dstp, ewp, zin)
    dinv, d2, h10, h11, hp1 = _tc_prep(degp.reshape(NW, 800, 128), e0, e1, e2, W1)

    s1 = _sc_prop(srcp, dstp, ewp, hp1.reshape(-1), zin).reshape(2, 2, 800, 128)
    h20, h21, hp2 = _tc_mid(s1, dinv, d2, h10, h11, b1.reshape(1, 2), W2)

    s2 = _sc_prop(srcp, dstp, ewp, hp2.reshape(-1), zin).reshape(2, 2, 800, 128)
    x30, x31 = _tc_post(s2, dinv, d2, h20, h21, b2.reshape(1, 2))

    gh0, gh1, ga0, ga1 = _sc_pairs(x30.reshape(-1), x31.reshape(-1),
                                   home.astype(_i32), away.astype(_i32))
    o0, o1, o2 = _tc_head(gh0.reshape(128, 128), gh1.reshape(128, 128),
                          ga0.reshape(128, 128), ga1.reshape(128, 128),
                          Wl1, bl1.reshape(1, 6), Wl3, bl3.reshape(1, 3))
    return jnp.stack(
        [o0.reshape(-1), o1.reshape(-1), o2.reshape(-1)], axis=-1)


# speedup vs baseline: 2.5665x; 2.5665x over previous
import dataclasses
import functools

import jax
import jax.numpy as jnp
from jax import lax
from jax.experimental import pallas as pl
from jax.experimental.pallas import tpu as pltpu
from jax.experimental.pallas import tpu_sc as plsc

N = 100000
E = 3200000
B = 16384

N_PAD = 102400
NC, NS = 2, 16
NW = NC * NS
EPT = E // NW
CH = 800
NCHUNK = EPT // CH
SLICE = N_PAD // NS
BPW = B // NW

_mesh = plsc.VectorSubcoreMesh(core_axis_name="c", subcore_axis_name="s")
_f32 = jnp.float32
_i32 = jnp.int32

_cp = pltpu.CompilerParams()
if "needs_layout_passes" in pltpu.CompilerParams.__dataclass_fields__:
    _cp = dataclasses.replace(_cp, needs_layout_passes=False)


def _wrap(chunk):
    return jnp.where(chunk >= NCHUNK, chunk - NCHUNK, chunk)


@functools.partial(
    pl.kernel,
    compiler_params=_cp,
    out_type=jax.ShapeDtypeStruct((NC, N_PAD), _f32),
    mesh=_mesh,
    scratch_types=(
        [pltpu.VMEM_SHARED((N_PAD,), _f32)]
        + [pltpu.VMEM((CH,), _i32) for _ in range(4)]
        + [pltpu.VMEM((CH,), _f32) for _ in range(4)]
        + [pltpu.SemaphoreType.DMA for _ in range(8)]
    ),
)
def _sc_deg(ei_hbm, ew_hbm, z_hbm, out_hbm, acc,
            ix0, ix1, ix2, ix3, v0, v1, v2, v3,
            sx0, sx1, sx2, sx3, ss0, ss1, ss2, ss3):
    c = lax.axis_index("c")
    s = lax.axis_index("s")
    w = c * NS + s
    IX = (ix0, ix1, ix2, ix3)
    V = (v0, v1, v2, v3)
    SX = (sx0, sx1, sx2, sx3)
    SS = (ss0, ss1, ss2, ss3)
    ebase = w * EPT

    def issue_in(chunk, bk):
        off = ebase + _wrap(chunk) * CH
        pltpu.async_copy(ei_hbm.at[pl.ds(E + off, CH)], IX[bk], SX[bk])
        pltpu.async_copy(ew_hbm.at[pl.ds(off, CH)], V[bk], SX[bk])

    def wait_in(chunk, bk):
        off = ebase + _wrap(chunk) * CH
        pltpu.make_async_copy(ei_hbm.at[pl.ds(E + off, CH)], IX[bk], SX[bk]).wait()
        pltpu.make_async_copy(ew_hbm.at[pl.ds(off, CH)], V[bk], SX[bk]).wait()

    def drain_sc(bk):
        pltpu.make_async_copy(V[bk], acc.at[IX[bk]], SS[bk]).wait()

    def process(chunk, bk, drain):
        wait_in(chunk, bk)
        if drain:
            drain_sc((bk + 2) % 4)
        issue_in(chunk + 2, (bk + 2) % 4)
        pltpu.async_copy(V[bk], acc.at[IX[bk]], SS[bk], add=True)

    issue_in(0, 0)
    issue_in(1, 1)
    pltpu.sync_copy(z_hbm.at[pl.ds(s * SLICE, SLICE)],
                    acc.at[pl.ds(s * SLICE, SLICE)])
    plsc.subcore_barrier()
    process(0, 0, False)
    process(1, 1, False)

    @pl.loop(2, NCHUNK - 3, step=4)
    def _(i0):
        for b in range(4):
            process(i0 + b, (2 + b) % 4, True)

    process(122, 2, True)
    process(123, 3, True)
    process(124, 0, True)
    drain_sc(3)
    drain_sc(0)
    wait_in(125, 1)
    wait_in(126, 2)
    plsc.subcore_barrier()
    pltpu.sync_copy(acc.at[pl.ds(s * SLICE, SLICE)],
                    out_hbm.at[c, pl.ds(s * SLICE, SLICE)])


@functools.partial(
    pl.kernel,
    compiler_params=_cp,
    out_type=jax.ShapeDtypeStruct((NC, 2, N_PAD), _f32),
    mesh=_mesh,
    scratch_types=(
        [pltpu.VMEM((N_PAD,), _i32)]
        + [pltpu.VMEM_SHARED((N_PAD,), _f32) for _ in range(2)]
        + [pltpu.VMEM((CH,), _i32) for _ in range(2)]
        + [pltpu.VMEM((CH,), _f32) for _ in range(2)]
        + [pltpu.VMEM((CH,), _i32) for _ in range(4)]
        + [pltpu.VMEM((CH,), _f32) for _ in range(4)]
        + [pltpu.SemaphoreType.DMA for _ in range(8)]
    ),
)
def _sc_prop(ei_hbm, ew_hbm, hp_hbm, z_hbm, out_hbm,
             tab, acc0, acc1, sb0, sb1, wb0, wb1, ix0, ix1, ix2, ix3,
             m00, m01, m10, m11,
             si0, si1, sx0, sx1, sx2, sx3, ss0, ss1):
    c = lax.axis_index("c")
    s = lax.axis_index("s")
    w = c * NS + s
    SB = (sb0, sb1)
    WB = (wb0, wb1)
    IX = (ix0, ix1, ix2, ix3)
    M0 = (m00, m01)
    M1 = (m10, m11)
    SI = (si0, si1)
    SX = (sx0, sx1, sx2, sx3)
    SS = (ss0, ss1)
    ebase = w * EPT

    def issue_in(chunk, bin_, bix):
        off = ebase + _wrap(chunk) * CH
        pltpu.async_copy(ei_hbm.at[pl.ds(off, CH)], SB[bin_], SI[bin_])
        pltpu.async_copy(ew_hbm.at[pl.ds(off, CH)], WB[bin_], SI[bin_])
        pltpu.async_copy(ei_hbm.at[pl.ds(E + off, CH)], IX[bix], SX[bix])

    def wait_in(chunk, bin_, bix):
        off = ebase + _wrap(chunk) * CH
        pltpu.make_async_copy(ei_hbm.at[pl.ds(off, CH)], SB[bin_], SI[bin_]).wait()
        pltpu.make_async_copy(ew_hbm.at[pl.ds(off, CH)], WB[bin_], SI[bin_]).wait()
        pltpu.make_async_copy(ei_hbm.at[pl.ds(E + off, CH)], IX[bix], SX[bix]).wait()

    def drain_sc(bin_, bix):
        pltpu.make_async_copy(M0[bin_], acc0.at[IX[bix]], SS[bin_]).wait()
        pltpu.make_async_copy(M1[bin_], acc1.at[IX[bix]], SS[bin_]).wait()

    def issue_sc(bin_, bix):
        pltpu.async_copy(M0[bin_], acc0.at[IX[bix]], SS[bin_], add=True)
        pltpu.async_copy(M1[bin_], acc1.at[IX[bix]], SS[bin_], add=True)

    def process(chunk, bin_, bix, drain):
        wait_in(chunk, bin_, bix)
        if drain:
            drain_sc(bin_, (bix + 2) % 4)
        sb, wb, m0, m1 = SB[bin_], WB[bin_], M0[bin_], M1[bin_]

        @pl.loop(0, CH, step=16)
        def _(g):
            gi = plsc.load_gather(tab, [sb[pl.ds(g, 16)]])
            w16 = wb[pl.ds(g, 16)]
            m0[pl.ds(g, 16)] = plsc.bitcast(
                lax.shift_left(gi, 16), _f32) * w16
            m1[pl.ds(g, 16)] = plsc.bitcast(
                lax.bitwise_and(gi, jnp.int32(-65536)), _f32) * w16

        issue_in(chunk + 2, bin_, (bix + 2) % 4)
        issue_sc(bin_, bix)

    issue_in(0, 0, 0)
    issue_in(1, 1, 1)
    pltpu.sync_copy(z_hbm.at[pl.ds(s * SLICE, SLICE)],
                    acc0.at[pl.ds(s * SLICE, SLICE)])
    pltpu.sync_copy(z_hbm.at[pl.ds(s * SLICE, SLICE)],
                    acc1.at[pl.ds(s * SLICE, SLICE)])
    pltpu.sync_copy(hp_hbm, tab)
    plsc.subcore_barrier()
    process(0, 0, 0, False)
    process(1, 1, 1, False)

    @pl.loop(2, NCHUNK - 3, step=4)
    def _(i0):
        for b in range(4):
            process(i0 + b, b & 1, (2 + b) % 4, True)

    process(122, 0, 2, True)
    process(123, 1, 3, True)
    process(124, 0, 0, True)
    drain_sc(1, 3)
    drain_sc(0, 0)
    wait_in(125, 1, 1)
    wait_in(126, 0, 2)
    plsc.subcore_barrier()
    pltpu.sync_copy(acc0.at[pl.ds(s * SLICE, SLICE)],
                    out_hbm.at[c, 0, pl.ds(s * SLICE, SLICE)])
    pltpu.sync_copy(acc1.at[pl.ds(s * SLICE, SLICE)],
                    out_hbm.at[c, 1, pl.ds(s * SLICE, SLICE)])


@functools.partial(
    pl.kernel,
    compiler_params=_cp,
    out_type=[jax.ShapeDtypeStruct((B,), _f32) for _ in range(4)],
    mesh=_mesh,
    scratch_types=(
        [pltpu.VMEM((BPW,), _i32) for _ in range(2)]
        + [pltpu.VMEM((BPW,), _f32) for _ in range(4)]
        + [pltpu.SemaphoreType.DMA for _ in range(2)]
    ),
)
def _sc_pairs(x0_hbm, x1_hbm, home_hbm, away_hbm,
              oh0_hbm, oh1_hbm, oa0_hbm, oa1_hbm,
              hbuf, abuf, g0, g1, g2, g3, sem_i, sem_g):
    c = lax.axis_index("c")
    s = lax.axis_index("s")
    w = c * NS + s
    sl = pl.ds(w * BPW, BPW)
    pltpu.async_copy(home_hbm.at[sl], hbuf, sem_i)
    pltpu.async_copy(away_hbm.at[sl], abuf, sem_i)
    pltpu.make_async_copy(home_hbm.at[sl], hbuf, sem_i).wait()
    pltpu.make_async_copy(away_hbm.at[sl], abuf, sem_i).wait()
    pltpu.async_copy(x0_hbm.at[hbuf], g0, sem_g)
    pltpu.async_copy(x1_hbm.at[hbuf], g1, sem_g)
    pltpu.async_copy(x0_hbm.at[abuf], g2, sem_g)
    pltpu.async_copy(x1_hbm.at[abuf], g3, sem_g)
    pltpu.make_async_copy(x0_hbm.at[hbuf], g0, sem_g).wait()
    pltpu.make_async_copy(x1_hbm.at[hbuf], g1, sem_g).wait()
    pltpu.make_async_copy(x0_hbm.at[abuf], g2, sem_g).wait()
    pltpu.make_async_copy(x1_hbm.at[abuf], g3, sem_g).wait()
    pltpu.sync_copy(g0, oh0_hbm.at[sl])
    pltpu.sync_copy(g1, oh1_hbm.at[sl])
    pltpu.sync_copy(g2, oa0_hbm.at[sl])
    pltpu.sync_copy(g3, oa1_hbm.at[sl])


def _lrelu(x):
    return jnp.where(x >= 0, x, 0.01 * x)


def _pack_pair(h0, h1):
    u0 = lax.bitcast_convert_type(h0.astype(jnp.bfloat16), jnp.uint16)
    u1 = lax.bitcast_convert_type(h1.astype(jnp.bfloat16), jnp.uint16)
    p = u0.astype(jnp.uint32) | (u1.astype(jnp.uint32) << 16)
    return lax.bitcast_convert_type(p, jnp.int32)


def _tc_prep_body(degp_ref, e0_ref, e1_ref, e2_ref, w1_ref,
                  dinv_ref, d2_ref, h0_ref, h1_ref, hp_ref):
    degp = degp_ref[...]
    deg = degp[0] + degp[1] + 1.0
    dv = lax.rsqrt(deg)
    dinv_ref[...] = dv
    d2_ref[...] = dv * dv
    w1 = w1_ref[...]
    e0, e1, e2 = e0_ref[...], e1_ref[...], e2_ref[...]
    h0 = e0 * w1[0, 0] + e1 * w1[1, 0] + e2 * w1[2, 0]
    h1 = e0 * w1[0, 1] + e1 * w1[1, 1] + e2 * w1[2, 1]
    h0_ref[...] = h0
    h1_ref[...] = h1
    hp_ref[...] = _pack_pair(dv * h0, dv * h1)


def _tc_prep(degp, e0, e1, e2, w1):
    shp = jax.ShapeDtypeStruct((800, 128), _f32)
    shpi = jax.ShapeDtypeStruct((800, 128), _i32)
    return pl.pallas_call(
        _tc_prep_body, out_shape=[shp, shp, shp, shp, shpi])(degp, e0, e1, e2, w1)


def _tc_mid_body(sp_ref, dinv_ref, d2_ref, h0_ref, h1_ref, b_ref, w2_ref,
                 o0_ref, o1_ref, hp_ref):
    sp = sp_ref[...]
    dv = dinv_ref[...]
    d2 = d2_ref[...]
    b = b_ref[...]
    x0 = _lrelu(dv * (sp[0, 0] + sp[1, 0]) + d2 * h0_ref[...] + b[0, 0])
    x1 = _lrelu(dv * (sp[0, 1] + sp[1, 1]) + d2 * h1_ref[...] + b[0, 1])
    w2 = w2_ref[...]
    h20 = x0 * w2[0, 0] + x1 * w2[1, 0]
    h21 = x0 * w2[0, 1] + x1 * w2[1, 1]
    o0_ref[...] = h20
    o1_ref[...] = h21
    hp_ref[...] = _pack_pair(dv * h20, dv * h21)


def _tc_mid(sp, dinv, d2, h0, h1, b1, w2):
    shp = jax.ShapeDtypeStruct((800, 128), _f32)
    shpi = jax.ShapeDtypeStruct((800, 128), _i32)
    return pl.pallas_call(
        _tc_mid_body, out_shape=[shp, shp, shpi])(sp, dinv, d2, h0, h1, b1, w2)


def _tc_post_body(sp_ref, dinv_ref, d2_ref, h0_ref, h1_ref, b_ref,
                  o0_ref, o1_ref):
    sp = sp_ref[...]
    dv = dinv_ref[...]
    d2 = d2_ref[...]
    b = b_ref[...]
    o0_ref[...] = _lrelu(dv * (sp[0, 0] + sp[1, 0]) + d2 * h0_ref[...] + b[0, 0])
    o1_ref[...] = _lrelu(dv * (sp[0, 1] + sp[1, 1]) + d2 * h1_ref[...] + b[0, 1])


def _tc_post(sp, dinv, d2, h0, h1, b2):
    shp = jax.ShapeDtypeStruct((800, 128), _f32)
    return pl.pallas_call(
        _tc_post_body, out_shape=[shp, shp])(sp, dinv, d2, h0, h1, b2)


def _tc_head_body(gh0_ref, gh1_ref, ga0_ref, ga1_ref,
                  wl1_ref, bl1_ref, wl3_ref, bl3_ref, o0_ref, o1_ref, o2_ref):
    xs = (gh0_ref[...], gh1_ref[...], ga0_ref[...], ga1_ref[...])
    wl1, bl1 = wl1_ref[...], bl1_ref[...]
    ys = []
    for m in range(6):
        acc = bl1[0, m]
        for k in range(4):
            acc = acc + xs[k] * wl1[k, m]
        ys.append(_lrelu(acc))
    wl3, bl3 = wl3_ref[...], bl3_ref[...]
    outs = (o0_ref, o1_ref, o2_ref)
    for t in range(3):
        acc = bl3[0, t]
        for m in range(6):
            acc = acc + ys[m] * wl3[m, t]
        y = _lrelu(acc)
        mx = jnp.max(y)
        lse = jnp.log(jnp.sum(jnp.exp(y - mx))) + mx
        outs[t][...] = y - lse


def _tc_head(gh0, gh1, ga0, ga1, wl1, bl1, wl3, bl3):
    shp = jax.ShapeDtypeStruct((128, 128), _f32)
    return pl.pallas_call(
        _tc_head_body,
        out_shape=[shp, shp, shp])(gh0, gh1, ga0, ga1, wl1, bl1, wl3, bl3)


def kernel(edge_index, edge_weight, home, away, emb,
           W1, b1, W2, b2, Wl1, bl1, Wl3, bl3):
    ei = edge_index.astype(_i32).reshape(-1)
    ew = edge_weight.astype(_f32)
    zin = jnp.zeros((N_PAD,), _f32)

    embp = jnp.pad(emb.astype(_f32), ((0, N_PAD - N), (0, 0)))
    e0 = embp[:, 0].reshape(800, 128)
    e1 = embp[:, 1].reshape(800, 128)
    e2 = embp[:, 2].reshape(800, 128)

    degp = _sc_deg(ei, ew, zin)
    dinv, d2, h10, h11, hp1 = _tc_prep(degp.reshape(2, 800, 128), e0, e1, e2, W1)

    s1 = _sc_prop(ei, ew, hp1.reshape(-1), zin).reshape(2, 2, 800, 128)
    h20, h21, hp2 = _tc_mid(s1, dinv, d2, h10, h11, b1.reshape(1, 2), W2)

    s2 = _sc_prop(ei, ew, hp2.reshape(-1), zin).reshape(2, 2, 800, 128)
    x30, x31 = _tc_post(s2, dinv, d2, h20, h21, b2.reshape(1, 2))

    gh0, gh1, ga0, ga1 = _sc_pairs(x30.reshape(-1), x31.reshape(-1),
                                   home.astype(_i32), away.astype(_i32))
    o0, o1, o2 = _tc_head(gh0.reshape(128, 128), gh1.reshape(128, 128),
                          ga0.reshape(128, 128), ga1.reshape(128, 128),
                          Wl1, bl1.reshape(1, 6), Wl3, bl3.reshape(1, 3))
    return jnp.stack(
        [o0.reshape(-1), o1.reshape(-1), o2.reshape(-1)], axis=-1)
